# speedup vs baseline: 2.8248x; 2.8248x over previous
import jax
import jax.numpy as jnp
from jax.experimental import pallas as pl


def _posembed_add_kernel(x_ref, w_ref, o_ref):
    o_ref[0] = x_ref[0] + w_ref[...]


def kernel(inputs, embedding_weight):
    B, S, D = inputs.shape
    S_BLK = 512
    return pl.pallas_call(
        _posembed_add_kernel,
        grid=(S // S_BLK, B),
        in_specs=[
            pl.BlockSpec((1, S_BLK, D), lambda s, b: (b, s, 0)),
            pl.BlockSpec((S_BLK, D), lambda s, b: (s, 0)),
        ],
        out_specs=pl.BlockSpec((1, S_BLK, D), lambda s, b: (b, s, 0)),
        out_shape=jax.ShapeDtypeStruct((B, S, D), inputs.dtype),
    )(inputs, embedding_weight)


# speedup vs baseline: 3.2844x; 1.1627x over previous
import jax
import jax.numpy as jnp
from jax.experimental import pallas as pl


def _posembed_add_kernel(x_ref, w_ref, o_ref):
    o_ref[...] = x_ref[...] + w_ref[...][None]


def kernel(inputs, embedding_weight):
    B, S, D = inputs.shape
    S_BLK = 512
    return pl.pallas_call(
        _posembed_add_kernel,
        grid=(S // S_BLK,),
        in_specs=[
            pl.BlockSpec((B, S_BLK, D), lambda s: (0, s, 0)),
            pl.BlockSpec((S_BLK, D), lambda s: (s, 0)),
        ],
        out_specs=pl.BlockSpec((B, S_BLK, D), lambda s: (0, s, 0)),
        out_shape=jax.ShapeDtypeStruct((B, S, D), inputs.dtype),
    )(inputs, embedding_weight)
